# baseline (speedup 1.0000x reference)
import functools

import jax
import jax.numpy as jnp
from jax import lax
from jax.experimental import pallas as pl
from jax.experimental.pallas import tpu as pltpu
from jax.experimental.pallas import tpu_sc as plsc

B = 16384
L = 50
D = 64
NC, NS = 2, 16
NW = NC * NS
CB = 16
ROWS = CB * L
CHUNKS = B // (NW * CB)
VL = 16
NIDX = 128
NST = ROWS // NIDX
REM = ROWS - NST * NIDX


def _phrase_kernel(idx_hbm, table_hbm, out_hbm, idx0, idx1, buf0, buf1,
                   outb0, outb1, sem0, sem1, osem):
    wid = lax.axis_index("s") * NC + lax.axis_index("c")
    chunk0 = wid * CHUNKS
    zero = jnp.zeros((VL,), jnp.float32)
    inv_l = jnp.float32(1.0 / L)

    def fire(g, idx_v, sem, buf):
        pltpu.sync_copy(idx_hbm.at[pl.ds(g * ROWS, ROWS)], idx_v)
        for s in range(NST):
            pltpu.async_copy(
                table_hbm.at[idx_v.at[pl.ds(s * NIDX, NIDX)]],
                buf.at[pl.ds(s * NIDX, NIDX)], sem)
        if REM:
            pltpu.async_copy(
                table_hbm.at[idx_v.at[pl.ds(NST * NIDX, REM)]],
                buf.at[pl.ds(NST * NIDX, REM)], sem)

    def drain(idx_v, sem, buf):
        pltpu.make_async_copy(table_hbm.at[idx_v], buf, sem).wait()

    def reduce(g, buf, ob):
        for p in range(CB):
            base = p * L

            def body(j, acc):
                return tuple(
                    acc[c] + buf[base + j, pl.ds(c * VL, VL)]
                    for c in range(D // VL)
                )

            acc = lax.fori_loop(0, L, body, (zero,) * (D // VL), unroll=10)
            for c in range(D // VL):
                ob[p, pl.ds(c * VL, VL)] = acc[c] * inv_l
        pltpu.async_copy(ob, out_hbm.at[pl.ds(g * CB, CB)], osem)

    def out_drain(ob):
        pltpu.make_async_copy(ob, out_hbm.at[pl.ds(0, CB)], osem).wait()

    fire(chunk0, idx0, sem0, buf0)

    def step(t, carry):
        g = chunk0 + 2 * t
        fire(g + 1, idx1, sem1, buf1)
        drain(idx0, sem0, buf0)

        @pl.when(t > 0)
        def _():
            out_drain(outb0)

        reduce(g, buf0, outb0)

        @pl.when(t < CHUNKS // 2 - 1)
        def _():
            fire(g + 2, idx0, sem0, buf0)

        drain(idx1, sem1, buf1)

        @pl.when(t > 0)
        def _():
            out_drain(outb1)

        reduce(g + 1, buf1, outb1)
        return carry

    lax.fori_loop(0, CHUNKS // 2, step, 0)
    out_drain(outb0)
    out_drain(outb1)


def kernel(indices, table):
    idx = indices.astype(jnp.int32).reshape(B * L)
    mesh = plsc.VectorSubcoreMesh(core_axis_name="c", subcore_axis_name="s")
    run = functools.partial(
        pl.kernel,
        out_type=jax.ShapeDtypeStruct((B, D), jnp.float32),
        mesh=mesh,
        compiler_params=pltpu.CompilerParams(use_tc_tiling_on_sc=False),
        scratch_types=[
            pltpu.VMEM((ROWS,), jnp.int32),
            pltpu.VMEM((ROWS,), jnp.int32),
            pltpu.VMEM((ROWS, D), jnp.float32),
            pltpu.VMEM((ROWS, D), jnp.float32),
            pltpu.VMEM((CB, D), jnp.float32),
            pltpu.VMEM((CB, D), jnp.float32),
            pltpu.SemaphoreType.DMA,
            pltpu.SemaphoreType.DMA,
            pltpu.SemaphoreType.DMA,
        ],
    )(_phrase_kernel)
    return run(idx, table)
